# baseline (speedup 1.0000x reference)
import functools

import jax
import jax.numpy as jnp
from jax import lax
from jax.experimental import pallas as pl
from jax.experimental.pallas import tpu as pltpu
from jax.experimental.pallas import tpu_sc as plsc

B = 4096
T = 50
EMB = 128
N_EXT = 5
OUT = 3 * EMB + N_EXT
R = B * T

NC = 2
NS = 16
NW = NC * NS
B_W = B // NW
NB = 2
C = NB * T
NCHUNK = B_W // NB


def _sc_kernel_body(idxa_h, idxp_h, idxs_h, ext_h, ta_h, tp_h, ts_h, out_h,
                    idxa, idxp, idxs, rows0, rows1, rows2, exts, extd,
                    sem_g0, sem_g1, sem_g2, sem_w0, sem_w1, sem_w2, sem_we):
  wid = lax.axis_index("s") * NC + lax.axis_index("c")
  b0_w = wid * B_W
  iota = lax.iota(jnp.int32, 16)
  rows = (rows0, rows1, rows2)
  sem_g = (sem_g0, sem_g1, sem_g2)
  sem_w = (sem_w0, sem_w1, sem_w2)
  idx_all = (idxa, idxp, idxs)
  tabs = (ta_h, tp_h, ts_h)

  pltpu.sync_copy(idxa_h.at[pl.ds(b0_w, B_W)], idxa)
  pltpu.sync_copy(idxp_h.at[pl.ds(b0_w, B_W)], idxp)
  pltpu.sync_copy(idxs_h.at[pl.ds(b0_w, B_W)], idxs)

  def band(b0, t):
    return out_h.at[pl.ds(b0, NB), :, pl.ds(t * EMB, EMB)]

  def extra_band(b):
    return out_h.at[b, :, pl.ds(3 * EMB, N_EXT)]

  def body(ci, carry):
    b0 = b0_w + ci * NB

    gathers = []
    for t in range(3):
      @pl.when(ci > 0)
      def _(t=t):
        pltpu.make_async_copy(rows[t], band(b0_w, t), sem_w[t]).wait()
      for k in range(NB):
        gathers.append(pltpu.async_copy(
            tabs[t].at[idx_all[t].at[ci * NB + k]], rows[t].at[k], sem_g[t]))

    @pl.when(ci > 0)
    def _():
      for bb in range(NB):
        pltpu.make_async_copy(
            extd.at[pl.ds(bb * T, T)], extra_band(b0_w + bb), sem_we).wait()
    parity = lax.rem(ci, 2)

    @pl.when(parity == 0)
    def _():
      off = pl.multiple_of(b0 * (N_EXT * T), 2 * C * N_EXT)
      pltpu.sync_copy(ext_h.at[pl.ds(off, 2 * C * N_EXT)], exts)
    half = parity * (C * N_EXT)
    for bb in range(NB):
      for j in range(N_EXT):
        for t0 in (0, 16, 32, 34):
          src = half + jnp.int32(bb * (N_EXT * T) + j * T + t0) + iota
          v = plsc.load_gather(exts, [src])
          r = jnp.int32(bb * T + t0) + iota
          plsc.store_scatter(extd, [r, jnp.full((16,), j, jnp.int32)], v)
    for bb in range(NB):
      pltpu.async_copy(extd.at[pl.ds(bb * T, T)], extra_band(b0 + bb), sem_we)

    for t in range(3):
      for k in range(NB):
        gathers[t * NB + k].wait()
      pltpu.async_copy(rows[t], band(b0, t), sem_w[t])
    return carry

  lax.fori_loop(0, NCHUNK, body, 0)
  for t in range(3):
    pltpu.make_async_copy(rows[t], band(b0_w, t), sem_w[t]).wait()
  for bb in range(NB):
    pltpu.make_async_copy(
        extd.at[pl.ds(bb * T, T)], extra_band(b0_w + bb), sem_we).wait()


BT = 128
OUT_PAD = 392


def _tc_transpose_body(y_ref, z_ref):
  eye = (lax.broadcasted_iota(jnp.int32, (BT, BT), 0) ==
         lax.broadcasted_iota(jnp.int32, (BT, BT), 1)).astype(jnp.float32)
  for t in range(T):
    x = y_ref[:, t, :]
    z_ref[t, :OUT, :] = lax.dot_general(
        x, eye, (((0,), (0,)), ((), ())),
        precision=lax.Precision.HIGHEST,
        preferred_element_type=jnp.float32)


def _tc_transpose(y):
  return pl.pallas_call(
      _tc_transpose_body,
      grid=(B // BT,),
      in_specs=[pl.BlockSpec((BT, T, OUT), lambda i: (i, 0, 0))],
      out_specs=pl.BlockSpec((T, OUT_PAD, BT), lambda i: (0, 0, i)),
      out_shape=jax.ShapeDtypeStruct((T, OUT_PAD, B), jnp.float32),
      compiler_params=pltpu.CompilerParams(
          vmem_limit_bytes=100 * 1024 * 1024),
  )(y)


@jax.jit
def _run(idxa, idxp, idxs, ext, ta, tp, ts):
  mesh = plsc.VectorSubcoreMesh(core_axis_name="c", subcore_axis_name="s")
  f = pl.kernel(
      _sc_kernel_body,
      out_type=jax.ShapeDtypeStruct((B, T, OUT), jnp.float32),
      mesh=mesh,
      compiler_params=pltpu.CompilerParams(needs_layout_passes=False),
      scratch_types=[
          pltpu.VMEM((B_W, T), jnp.int32),
          pltpu.VMEM((B_W, T), jnp.int32),
          pltpu.VMEM((B_W, T), jnp.int32),
          pltpu.VMEM((NB, T, EMB), jnp.float32),
          pltpu.VMEM((NB, T, EMB), jnp.float32),
          pltpu.VMEM((NB, T, EMB), jnp.float32),
          pltpu.VMEM((2 * C * N_EXT,), jnp.float32),
          pltpu.VMEM((C, N_EXT), jnp.float32),
          pltpu.SemaphoreType.DMA,
          pltpu.SemaphoreType.DMA,
          pltpu.SemaphoreType.DMA,
          pltpu.SemaphoreType.DMA,
          pltpu.SemaphoreType.DMA,
          pltpu.SemaphoreType.DMA,
          pltpu.SemaphoreType.DMA,
      ],
  )
  y = f(idxa, idxp, idxs, ext, ta, tp, ts)
  return _tc_transpose(y)


def kernel(obs, action_embeddings, parent_embeddings, sibling_embeddings):
  idxa = obs[:, 0, :].astype(jnp.int32)
  idxp = obs[:, 1, :].astype(jnp.int32)
  idxs = obs[:, 2, :].astype(jnp.int32)
  ext = obs[:, 3:, :].reshape(B * N_EXT * T)
  out_p = _run(idxa, idxp, idxs, ext, action_embeddings, parent_embeddings,
               sibling_embeddings)
  return jnp.transpose(out_p, (2, 0, 1))[:, :, :OUT]


# speedup vs baseline: 1.4512x; 1.4512x over previous
import functools

import jax
import jax.numpy as jnp
from jax import lax
from jax.experimental import pallas as pl
from jax.experimental.pallas import tpu as pltpu
from jax.experimental.pallas import tpu_sc as plsc

B = 4096
T = 50
EMB = 128
N_EXT = 5
OUT = 3 * EMB + N_EXT
R = B * T

NC = 2
NS = 16
NW = NC * NS
B_W = B // NW
NB = 2
C = NB * T
NCHUNK = B_W // NB


def _sc_kernel_body(idxa_h, idxp_h, idxs_h, ext_h, ta_h, tp_h, ts_h, out_h,
                    idxa, idxp, idxs, rows0, rows1, rows2, exts, extd,
                    sem_g0, sem_g1, sem_g2, sem_w0, sem_w1, sem_w2, sem_we):
  wid = lax.axis_index("s") * NC + lax.axis_index("c")
  b0_w = wid * B_W
  iota = lax.iota(jnp.int32, 16)
  rows = (rows0, rows1, rows2)
  sem_g = (sem_g0, sem_g1, sem_g2)
  sem_w = (sem_w0, sem_w1, sem_w2)
  idx_all = (idxa, idxp, idxs)
  tabs = (ta_h, tp_h, ts_h)

  pltpu.sync_copy(idxa_h.at[pl.ds(b0_w, B_W)], idxa)
  pltpu.sync_copy(idxp_h.at[pl.ds(b0_w, B_W)], idxp)
  pltpu.sync_copy(idxs_h.at[pl.ds(b0_w, B_W)], idxs)

  def band(b0, t):
    return out_h.at[pl.ds(b0, NB), :, pl.ds(t * EMB, EMB)]

  def extra_band(b):
    return out_h.at[b, :, pl.ds(3 * EMB, N_EXT)]

  def body(ci, carry):
    b0 = b0_w + ci * NB

    gathers = []
    for t in range(3):
      @pl.when(ci > 0)
      def _(t=t):
        pltpu.make_async_copy(rows[t], band(b0_w, t), sem_w[t]).wait()
      for k in range(NB):
        gathers.append(pltpu.async_copy(
            tabs[t].at[idx_all[t].at[ci * NB + k]], rows[t].at[k], sem_g[t]))

    @pl.when(ci > 0)
    def _():
      for bb in range(NB):
        pltpu.make_async_copy(
            extd.at[pl.ds(bb * T, T)], extra_band(b0_w + bb), sem_we).wait()
    parity = lax.rem(ci, 2)

    @pl.when(parity == 0)
    def _():
      off = pl.multiple_of(b0 * (N_EXT * T), 2 * C * N_EXT)
      pltpu.sync_copy(ext_h.at[pl.ds(off, 2 * C * N_EXT)], exts)
    half = parity * (C * N_EXT)
    for bb in range(NB):
      for j in range(N_EXT):
        for t0 in (0, 16, 32, 34):
          src = half + jnp.int32(bb * (N_EXT * T) + j * T + t0) + iota
          v = plsc.load_gather(exts, [src])
          r = jnp.int32(bb * T + t0) + iota
          plsc.store_scatter(extd, [r, jnp.full((16,), j, jnp.int32)], v)
    for bb in range(NB):
      pltpu.async_copy(extd.at[pl.ds(bb * T, T)], extra_band(b0 + bb), sem_we)

    for t in range(3):
      for k in range(NB):
        gathers[t * NB + k].wait()
      pltpu.async_copy(rows[t], band(b0, t), sem_w[t])
    return carry

  lax.fori_loop(0, NCHUNK, body, 0)
  for t in range(3):
    pltpu.make_async_copy(rows[t], band(b0_w, t), sem_w[t]).wait()
  for bb in range(NB):
    pltpu.make_async_copy(
        extd.at[pl.ds(bb * T, T)], extra_band(b0_w + bb), sem_we).wait()


BT = 128
OUT_PAD = 392


def _tc_transpose_body(y_ref, z_ref):
  for t in range(T):
    z_ref[t, :OUT, :] = y_ref[:, t, :].T


def _tc_transpose(y):
  return pl.pallas_call(
      _tc_transpose_body,
      grid=(B // BT,),
      in_specs=[pl.BlockSpec((BT, T, OUT), lambda i: (i, 0, 0))],
      out_specs=pl.BlockSpec((T, OUT_PAD, BT), lambda i: (0, 0, i)),
      out_shape=jax.ShapeDtypeStruct((T, OUT_PAD, B), jnp.float32),
      compiler_params=pltpu.CompilerParams(
          vmem_limit_bytes=100 * 1024 * 1024),
  )(y)


@jax.jit
def _run(idxa, idxp, idxs, ext, ta, tp, ts):
  mesh = plsc.VectorSubcoreMesh(core_axis_name="c", subcore_axis_name="s")
  f = pl.kernel(
      _sc_kernel_body,
      out_type=jax.ShapeDtypeStruct((B, T, OUT), jnp.float32),
      mesh=mesh,
      compiler_params=pltpu.CompilerParams(needs_layout_passes=False),
      scratch_types=[
          pltpu.VMEM((B_W, T), jnp.int32),
          pltpu.VMEM((B_W, T), jnp.int32),
          pltpu.VMEM((B_W, T), jnp.int32),
          pltpu.VMEM((NB, T, EMB), jnp.float32),
          pltpu.VMEM((NB, T, EMB), jnp.float32),
          pltpu.VMEM((NB, T, EMB), jnp.float32),
          pltpu.VMEM((2 * C * N_EXT,), jnp.float32),
          pltpu.VMEM((C, N_EXT), jnp.float32),
          pltpu.SemaphoreType.DMA,
          pltpu.SemaphoreType.DMA,
          pltpu.SemaphoreType.DMA,
          pltpu.SemaphoreType.DMA,
          pltpu.SemaphoreType.DMA,
          pltpu.SemaphoreType.DMA,
          pltpu.SemaphoreType.DMA,
      ],
  )
  y = f(idxa, idxp, idxs, ext, ta, tp, ts)
  return _tc_transpose(y)


def kernel(obs, action_embeddings, parent_embeddings, sibling_embeddings):
  idxa = obs[:, 0, :].astype(jnp.int32)
  idxp = obs[:, 1, :].astype(jnp.int32)
  idxs = obs[:, 2, :].astype(jnp.int32)
  ext = obs[:, 3:, :].reshape(B * N_EXT * T)
  out_p = _run(idxa, idxp, idxs, ext, action_embeddings, parent_embeddings,
               sibling_embeddings)
  return jnp.transpose(out_p, (2, 0, 1))[:, :, :OUT]


# speedup vs baseline: 1.4564x; 1.0035x over previous
import functools

import jax
import jax.numpy as jnp
from jax import lax
from jax.experimental import pallas as pl
from jax.experimental.pallas import tpu as pltpu
from jax.experimental.pallas import tpu_sc as plsc

B = 4096
T = 50
EMB = 128
N_EXT = 5
OUT = 3 * EMB + N_EXT
R = B * T

NC = 2
NS = 16
NW = NC * NS
NH = 2
BH = B // NH
B_W = BH // NW
NB = 2
C = NB * T
NCHUNK = B_W // NB


def _make_sc_body(h):
  return functools.partial(_sc_kernel_body, h)


def _sc_kernel_body(h, idxa_h, idxp_h, idxs_h, ext_h, ta_h, tp_h, ts_h, out_h,
                    idxa, idxp, idxs, rows0, rows1, rows2, exts, extd,
                    sem_g0, sem_g1, sem_g2, sem_w0, sem_w1, sem_w2, sem_we):
  wid = lax.axis_index("s") * NC + lax.axis_index("c")
  b0_w = wid * B_W
  iota = lax.iota(jnp.int32, 16)
  rows = (rows0, rows1, rows2)
  sem_g = (sem_g0, sem_g1, sem_g2)
  sem_w = (sem_w0, sem_w1, sem_w2)
  idx_all = (idxa, idxp, idxs)
  tabs = (ta_h, tp_h, ts_h)

  gb_w = h * BH + b0_w
  pltpu.sync_copy(idxa_h.at[pl.ds(gb_w, B_W)], idxa)
  pltpu.sync_copy(idxp_h.at[pl.ds(gb_w, B_W)], idxp)
  pltpu.sync_copy(idxs_h.at[pl.ds(gb_w, B_W)], idxs)

  def band(b0, t):
    return out_h.at[pl.ds(b0, NB), :, pl.ds(t * EMB, EMB)]

  def extra_band(b):
    return out_h.at[b, :, pl.ds(3 * EMB, N_EXT)]

  def body(ci, carry):
    b0 = b0_w + ci * NB

    gathers = []
    for t in range(3):
      @pl.when(ci > 0)
      def _(t=t):
        pltpu.make_async_copy(rows[t], band(b0_w, t), sem_w[t]).wait()
      for k in range(NB):
        gathers.append(pltpu.async_copy(
            tabs[t].at[idx_all[t].at[ci * NB + k]], rows[t].at[k], sem_g[t]))

    @pl.when(ci > 0)
    def _():
      for bb in range(NB):
        pltpu.make_async_copy(
            extd.at[pl.ds(bb * T, T)], extra_band(b0_w + bb), sem_we).wait()
    parity = lax.rem(ci, 2)

    @pl.when(parity == 0)
    def _():
      off = pl.multiple_of((h * BH + b0) * (N_EXT * T), 2 * C * N_EXT)
      pltpu.sync_copy(ext_h.at[pl.ds(off, 2 * C * N_EXT)], exts)
    half = parity * (C * N_EXT)
    for bb in range(NB):
      for j in range(N_EXT):
        for t0 in (0, 16, 32, 34):
          src = half + jnp.int32(bb * (N_EXT * T) + j * T + t0) + iota
          v = plsc.load_gather(exts, [src])
          r = jnp.int32(bb * T + t0) + iota
          plsc.store_scatter(extd, [r, jnp.full((16,), j, jnp.int32)], v)
    for bb in range(NB):
      pltpu.async_copy(extd.at[pl.ds(bb * T, T)], extra_band(b0 + bb), sem_we)

    for t in range(3):
      for k in range(NB):
        gathers[t * NB + k].wait()
      pltpu.async_copy(rows[t], band(b0, t), sem_w[t])
    return carry

  lax.fori_loop(0, NCHUNK, body, 0)
  for t in range(3):
    pltpu.make_async_copy(rows[t], band(b0_w, t), sem_w[t]).wait()
  for bb in range(NB):
    pltpu.make_async_copy(
        extd.at[pl.ds(bb * T, T)], extra_band(b0_w + bb), sem_we).wait()


BT = 128
OUT_PAD = 392


def _tc_transpose_body(y_ref, z_ref):
  for t in range(T):
    z_ref[t, :OUT, :] = y_ref[:, t, :].T


def _tc_transpose_alias_body(y_ref, z_in_ref, z_ref):
  del z_in_ref
  _tc_transpose_body(y_ref, z_ref)


_TC_PARAMS = dict(
    out_shape=jax.ShapeDtypeStruct((T, OUT_PAD, B), jnp.float32),
    compiler_params=pltpu.CompilerParams(vmem_limit_bytes=100 * 1024 * 1024),
)


def _tc_transpose_half(y, z, h):
  nblk = BH // BT
  out_spec = pl.BlockSpec((T, OUT_PAD, BT), lambda i, h=h: (0, 0, h * nblk + i))
  in_spec = pl.BlockSpec((BT, T, OUT), lambda i: (i, 0, 0))
  if z is None:
    return pl.pallas_call(
        _tc_transpose_body, grid=(nblk,), in_specs=[in_spec],
        out_specs=out_spec, **_TC_PARAMS)(y)
  return pl.pallas_call(
      _tc_transpose_alias_body, grid=(nblk,),
      in_specs=[in_spec, pl.BlockSpec(memory_space=pl.ANY)],
      out_specs=out_spec, input_output_aliases={1: 0}, **_TC_PARAMS)(y, z)


@jax.jit
def _run(idxa, idxp, idxs, ext, ta, tp, ts):
  mesh = plsc.VectorSubcoreMesh(core_axis_name="c", subcore_axis_name="s")
  ys = []
  for h in range(NH):
    f = pl.kernel(
        _make_sc_body(h),
        out_type=jax.ShapeDtypeStruct((BH, T, OUT), jnp.float32),
        mesh=mesh,
        compiler_params=pltpu.CompilerParams(needs_layout_passes=False),
        scratch_types=[
            pltpu.VMEM((B_W, T), jnp.int32),
            pltpu.VMEM((B_W, T), jnp.int32),
            pltpu.VMEM((B_W, T), jnp.int32),
            pltpu.VMEM((NB, T, EMB), jnp.float32),
            pltpu.VMEM((NB, T, EMB), jnp.float32),
            pltpu.VMEM((NB, T, EMB), jnp.float32),
            pltpu.VMEM((2 * C * N_EXT,), jnp.float32),
            pltpu.VMEM((C, N_EXT), jnp.float32),
            pltpu.SemaphoreType.DMA,
            pltpu.SemaphoreType.DMA,
            pltpu.SemaphoreType.DMA,
            pltpu.SemaphoreType.DMA,
            pltpu.SemaphoreType.DMA,
            pltpu.SemaphoreType.DMA,
            pltpu.SemaphoreType.DMA,
        ],
    )
    ys.append(f(idxa, idxp, idxs, ext, ta, tp, ts))
  z = None
  for h in range(NH):
    z = _tc_transpose_half(ys[h], z, h)
  return z


def kernel(obs, action_embeddings, parent_embeddings, sibling_embeddings):
  idxa = obs[:, 0, :].astype(jnp.int32)
  idxp = obs[:, 1, :].astype(jnp.int32)
  idxs = obs[:, 2, :].astype(jnp.int32)
  ext = obs[:, 3:, :].reshape(B * N_EXT * T)
  out_p = _run(idxa, idxp, idxs, ext, action_embeddings, parent_embeddings,
               sibling_embeddings)
  return jnp.transpose(out_p, (2, 0, 1))[:, :, :OUT]
